# baseline (speedup 1.0000x reference)
import jax
import jax.numpy as jnp
from jax.experimental import pallas as pl
from jax.experimental.pallas import tpu as pltpu

B = 16384
V = 213
D_OUT = 10
NCHUNK = 1
CH = B // NCHUNK


def _body(xt_hbm, w_ref, b_ref, out_hbm, xbuf, obuf, insem, outsem):
    in_cps = []
    for k in range(NCHUNK):
        cp = pltpu.make_async_copy(
            xt_hbm.at[:, pl.ds(k * CH, CH)], xbuf.at[k], insem.at[k]
        )
        cp.start(priority=k % 2)
        in_cps.append(cp)
    for k in range(NCHUNK):
        in_cps[k].wait()
        obuf[:, pl.ds(k * CH, CH)] = (
            jnp.dot(w_ref[...], xbuf[k], preferred_element_type=jnp.float32)
            + b_ref[...]
        )
    ocp = pltpu.make_async_copy(obuf, out_hbm, outsem)
    ocp.start()
    ocp.wait()


def kernel(x, W, b):
    xt = x.T
    b2 = b.reshape(D_OUT, 1)
    outT = pl.pallas_call(
        _body,
        in_specs=[
            pl.BlockSpec(memory_space=pl.ANY),
            pl.BlockSpec((D_OUT, V), lambda: (0, 0)),
            pl.BlockSpec((D_OUT, 1), lambda: (0, 0)),
        ],
        out_specs=pl.BlockSpec(memory_space=pl.ANY),
        out_shape=jax.ShapeDtypeStruct((D_OUT, B), jnp.float32),
        scratch_shapes=[
            pltpu.VMEM((NCHUNK, V, CH), jnp.float32),
            pltpu.VMEM((D_OUT, B), jnp.float32),
            pltpu.SemaphoreType.DMA((NCHUNK,)),
            pltpu.SemaphoreType.DMA,
        ],
    )(xt, W, b2)
    return outT.T


# speedup vs baseline: 1.0912x; 1.0912x over previous
import jax
import jax.numpy as jnp
from jax.experimental import pallas as pl
from jax.experimental.pallas import tpu as pltpu

B = 16384
V = 213
D_OUT = 10
NCHUNK = 4
CH = B // NCHUNK


def _body(xt_hbm, w_ref, b_ref, out_hbm, xbuf, obuf, insem, outsem):
    in_cps = []
    for k in range(NCHUNK):
        cp = pltpu.make_async_copy(
            xt_hbm.at[:, pl.ds(k * CH, CH)], xbuf.at[k], insem.at[k]
        )
        cp.start(priority=k % 2)
        in_cps.append(cp)
    out_cps = []
    for k in range(NCHUNK):
        in_cps[k].wait()
        obuf[:, pl.ds(k * CH, CH)] = (
            jnp.dot(w_ref[...], xbuf[k], preferred_element_type=jnp.float32)
            + b_ref[...]
        )
        ocp = pltpu.make_async_copy(
            obuf.at[:, pl.ds(k * CH, CH)],
            out_hbm.at[:, pl.ds(k * CH, CH)],
            outsem.at[k],
        )
        ocp.start(priority=(k + 1) % 2)
        out_cps.append(ocp)
    for k in range(NCHUNK):
        out_cps[k].wait()


def kernel(x, W, b):
    xt = x.T
    b2 = b.reshape(D_OUT, 1)
    outT = pl.pallas_call(
        _body,
        in_specs=[
            pl.BlockSpec(memory_space=pl.ANY),
            pl.BlockSpec((D_OUT, V), lambda: (0, 0)),
            pl.BlockSpec((D_OUT, 1), lambda: (0, 0)),
        ],
        out_specs=pl.BlockSpec(memory_space=pl.ANY),
        out_shape=jax.ShapeDtypeStruct((D_OUT, B), jnp.float32),
        scratch_shapes=[
            pltpu.VMEM((NCHUNK, V, CH), jnp.float32),
            pltpu.VMEM((D_OUT, B), jnp.float32),
            pltpu.SemaphoreType.DMA((NCHUNK,)),
            pltpu.SemaphoreType.DMA((NCHUNK,)),
        ],
    )(xt, W, b2)
    return outT.T
